# baseline (speedup 1.0000x reference)
import functools
import math

import jax
import jax.numpy as jnp
from jax import lax
from jax.experimental import pallas as pl
from jax.experimental.pallas import tpu as pltpu
from jax.experimental.pallas import tpu_sc as plsc

D_MODEL = 64
SCALE = math.sqrt(D_MODEL)
NUM_CORES = 2
NUM_SUBCORES = 16
NUM_WORKERS = NUM_CORES * NUM_SUBCORES
CHUNK = 512


@functools.partial(jax.jit, static_argnames=("batch",))
def _embed(tokens_flat, weight, *, batch):
    b_per_w = batch // NUM_WORKERS
    n_chunks = b_per_w // CHUNK
    mesh = plsc.VectorSubcoreMesh(core_axis_name="c", subcore_axis_name="s")

    @functools.partial(
        pl.kernel,
        mesh=mesh,
        out_type=jax.ShapeDtypeStruct((batch, D_MODEL), jnp.float32),
        scratch_types=[
            pltpu.VMEM((CHUNK,), jnp.int32),
            pltpu.VMEM((CHUNK, D_MODEL), jnp.float32),
            pltpu.SemaphoreType.DMA,
        ],
        compiler_params=pltpu.CompilerParams(use_tc_tiling_on_sc=False),
    )
    def emb_kernel(tok_hbm, w_hbm, out_hbm, idx_v, rows_v, sem):
        wid = lax.axis_index("s") * NUM_CORES + lax.axis_index("c")
        base = wid * b_per_w

        def chunk_body(ci, carry):
            off = base + ci * CHUNK
            pltpu.sync_copy(tok_hbm.at[pl.ds(off, CHUNK)], idx_v)
            pltpu.async_copy(w_hbm.at[idx_v], rows_v, sem).wait()

            def row_body(i, c2):
                for j in range(D_MODEL // 16):
                    sl = pl.ds(j * 16, 16)
                    rows_v[i, sl] = rows_v[i, sl] * SCALE
                return c2

            lax.fori_loop(0, CHUNK, row_body, 0, unroll=2)
            pltpu.sync_copy(rows_v, out_hbm.at[pl.ds(off, CHUNK)])
            return carry

        lax.fori_loop(0, n_chunks, chunk_body, 0)

    return emb_kernel(tokens_flat, weight)


def kernel(tokens, weight):
    n_seq, n_tok = tokens.shape
    batch = n_seq * n_tok
    tok_flat = tokens.reshape(batch).astype(jnp.int32)
    out = _embed(tok_flat, weight, batch=batch)
    return out.reshape(n_seq, n_tok, D_MODEL)


# speedup vs baseline: 1.0904x; 1.0904x over previous
import functools
import math

import jax
import jax.numpy as jnp
from jax import lax
from jax.experimental import pallas as pl
from jax.experimental.pallas import tpu as pltpu
from jax.experimental.pallas import tpu_sc as plsc

D_MODEL = 64
SCALE = math.sqrt(D_MODEL)
NUM_CORES = 2
NUM_SUBCORES = 16
NUM_WORKERS = NUM_CORES * NUM_SUBCORES
CHUNK = 256
NBUF = 4


@functools.partial(jax.jit, static_argnames=("batch",))
def _embed(tokens_flat, weight, *, batch):
    b_per_w = batch // NUM_WORKERS
    group = NBUF * CHUNK
    n_groups = b_per_w // group
    mesh = plsc.VectorSubcoreMesh(core_axis_name="c", subcore_axis_name="s")

    scratch = (
        [pltpu.VMEM((CHUNK,), jnp.int32) for _ in range(NBUF)]
        + [pltpu.VMEM((CHUNK, D_MODEL), jnp.float32) for _ in range(NBUF)]
        + [pltpu.SemaphoreType.DMA((NBUF,)), pltpu.SemaphoreType.DMA((NBUF,))]
    )

    @functools.partial(
        pl.kernel,
        mesh=mesh,
        out_type=jax.ShapeDtypeStruct((batch, D_MODEL), jnp.float32),
        scratch_types=scratch,
        compiler_params=pltpu.CompilerParams(use_tc_tiling_on_sc=False),
    )
    def emb_kernel(tok_hbm, w_hbm, out_hbm, *sc):
        idx_v = sc[:NBUF]
        rows_v = sc[NBUF:2 * NBUF]
        gsem, osem = sc[2 * NBUF], sc[2 * NBUF + 1]
        wid = lax.axis_index("s") * NUM_CORES + lax.axis_index("c")
        base = wid * b_per_w

        def group_body(g, carry):
            goff = base + g * group
            for b in range(NBUF):
                off = goff + b * CHUNK

                @pl.when(g > 0)
                def _drain():
                    pltpu.make_async_copy(
                        rows_v[b], out_hbm.at[pl.ds(off, CHUNK)], osem.at[b]
                    ).wait()

                pltpu.sync_copy(tok_hbm.at[pl.ds(off, CHUNK)], idx_v[b])
                pltpu.async_copy(w_hbm.at[idx_v[b]], rows_v[b], gsem.at[b])
            for b in range(NBUF):
                off = goff + b * CHUNK
                pltpu.make_async_copy(
                    w_hbm.at[idx_v[b]], rows_v[b], gsem.at[b]
                ).wait()

                def row_body(i, c2, _b=b):
                    for j in range(D_MODEL // 16):
                        sl = pl.ds(j * 16, 16)
                        rows_v[_b][i, sl] = rows_v[_b][i, sl] * SCALE
                    return c2

                lax.fori_loop(0, CHUNK, row_body, 0, unroll=4)
                pltpu.async_copy(rows_v[b], out_hbm.at[pl.ds(off, CHUNK)],
                                 osem.at[b])
            return carry

        lax.fori_loop(0, n_groups, group_body, 0)
        for b in range(NBUF):
            off = base + (n_groups - 1) * group + b * CHUNK
            pltpu.make_async_copy(
                rows_v[b], out_hbm.at[pl.ds(off, CHUNK)], osem.at[b]
            ).wait()

    return emb_kernel(tokens_flat, weight)


def kernel(tokens, weight):
    n_seq, n_tok = tokens.shape
    batch = n_seq * n_tok
    tok_flat = tokens.reshape(batch).astype(jnp.int32)
    out = _embed(tok_flat, weight, batch=batch)
    return out.reshape(n_seq, n_tok, D_MODEL)
